# Initial kernel scaffold; baseline (speedup 1.0000x reference)
import jax
import jax.numpy as jnp
from jax.experimental import pallas as pl


def kernel(x, edge_index, W1_0, W1_1, b1, W2_0, W2_1, b2):
    raise NotImplementedError("write your pallas kernel here")


# speedup vs baseline: 6.2818x; 6.2818x over previous
import dataclasses
import functools

import jax
import jax.numpy as jnp
from jax import lax
from jax.experimental import pallas as pl
from jax.experimental.pallas import tpu as pltpu
from jax.experimental.pallas import tpu_sc as plsc

NC = 2
NSUB = 16
L = 16
CHUNK = 128

_N = 10000
NACC = 10240
TRASH = _N
HALF = 128

BR = 2000


def _mesh():
    return plsc.VectorSubcoreMesh(
        core_axis_name="c", subcore_axis_name="s",
        num_cores=NC, num_subcores=NSUB)


def _sc_params():
    cp = pltpu.CompilerParams()
    if "needs_layout_passes" in pltpu.CompilerParams.__dataclass_fields__:
        cp = dataclasses.replace(cp, needs_layout_passes=False)
    return cp


def _sc_deg_colm(rowp, colp):
    ep = rowp.shape[0]
    per_w = ep // NSUB
    n_chunks = per_w // CHUNK
    rb = NACC // NSUB

    @functools.partial(
        pl.kernel,
        out_type=[jax.ShapeDtypeStruct((NACC, HALF), jnp.float32),
                  jax.ShapeDtypeStruct((ep,), jnp.int32)],
        mesh=_mesh(),
        compiler_params=_sc_params(),
        scratch_types=[
            pltpu.VMEM((CHUNK,), jnp.int32),
            pltpu.VMEM((CHUNK,), jnp.int32),
            pltpu.VMEM((CHUNK,), jnp.int32),
            pltpu.VMEM((CHUNK, HALF), jnp.float32),
            pltpu.VMEM_SHARED((NACC, HALF), jnp.float32),
        ],
    )
    def k(row_hbm, col_hbm, deg_hbm, colm_hbm,
          row_v, col_v, idxm_v, ones_v, degacc):
        c = lax.axis_index("c")
        s = lax.axis_index("s")

        @pl.loop(0, CHUNK)
        def _(i):
            for j in range(HALF // L):
                ones_v[i, pl.ds(j * L, L)] = jnp.zeros((L,), jnp.float32)

        @pl.when(c == 0)
        def _():
            @pl.loop(0, rb // CHUNK)
            def _(i):
                pltpu.sync_copy(ones_v,
                                degacc.at[pl.ds(s * rb + i * CHUNK, CHUNK)])

        @pl.loop(0, CHUNK)
        def _(i):
            for j in range(HALF // L):
                ones_v[i, pl.ds(j * L, L)] = jnp.full((L,), 1.0, jnp.float32)

        plsc.subcore_barrier()
        base_w = s * per_w

        @pl.loop(0, n_chunks)
        def _(t):
            base = base_w + t * CHUNK
            pltpu.sync_copy(row_hbm.at[pl.ds(base, CHUNK)], row_v)
            pltpu.sync_copy(col_hbm.at[pl.ds(base, CHUNK)], col_v)
            for j in range(CHUNK // L):
                sl = pl.ds(j * L, L)
                r = row_v[sl]
                q = col_v[sl]
                m = r == q

                @pl.when(c == 0)
                def _():
                    idxm_v[sl] = jnp.where(m, TRASH, r)

                @pl.when(c == 1)
                def _():
                    idxm_v[sl] = jnp.where(m, TRASH, q)

            @pl.when(c == 0)
            def _():
                pltpu.sync_copy(ones_v, degacc.at[idxm_v], add=True)

            @pl.when(c == 1)
            def _():
                pltpu.sync_copy(idxm_v, colm_hbm.at[pl.ds(base, CHUNK)])

        plsc.subcore_barrier()

        @pl.when(c == 0)
        def _():
            pltpu.sync_copy(degacc.at[pl.ds(s * rb, rb)],
                            deg_hbm.at[pl.ds(s * rb, rb)])

    return k(rowp, colp)


def _sc_prop(xs0, xs1, rowp, colm):
    ep = rowp.shape[0]
    per_w = ep // NSUB
    n_chunks = per_w // CHUNK
    rb = NACC // NSUB

    @functools.partial(
        pl.kernel,
        out_type=[jax.ShapeDtypeStruct((NACC, HALF), jnp.float32),
                  jax.ShapeDtypeStruct((NACC, HALF), jnp.float32)],
        mesh=_mesh(),
        compiler_params=_sc_params(),
        scratch_types=[
            pltpu.VMEM((CHUNK,), jnp.int32),
            pltpu.VMEM((CHUNK,), jnp.int32),
            pltpu.VMEM((CHUNK, HALF), jnp.float32),
            pltpu.VMEM((CHUNK, HALF), jnp.float32),
            pltpu.VMEM_SHARED((NACC, HALF), jnp.float32),
            pltpu.SemaphoreType.DMA,
        ],
    )
    def k(x0_hbm, x1_hbm, row_hbm, colm_hbm, o0_hbm, o1_hbm,
          row_v, colm_v, gbuf, zbuf, acc, sem):
        c = lax.axis_index("c")
        s = lax.axis_index("s")

        @pl.loop(0, CHUNK)
        def _(i):
            for j in range(HALF // L):
                zbuf[i, pl.ds(j * L, L)] = jnp.zeros((L,), jnp.float32)

        @pl.loop(0, rb // CHUNK)
        def _(i):
            pltpu.sync_copy(zbuf, acc.at[pl.ds(s * rb + i * CHUNK, CHUNK)])

        plsc.subcore_barrier()
        base_w = s * per_w

        @pl.loop(0, n_chunks)
        def _(t):
            base = base_w + t * CHUNK
            pltpu.sync_copy(row_hbm.at[pl.ds(base, CHUNK)], row_v)
            pltpu.sync_copy(colm_hbm.at[pl.ds(base, CHUNK)], colm_v)

            @pl.when(c == 0)
            def _():
                pltpu.async_copy(x0_hbm.at[row_v], gbuf, sem).wait()

            @pl.when(c == 1)
            def _():
                pltpu.async_copy(x1_hbm.at[row_v], gbuf, sem).wait()

            pltpu.sync_copy(gbuf, acc.at[colm_v], add=True)

        plsc.subcore_barrier()

        @pl.when(c == 0)
        def _():
            pltpu.sync_copy(acc.at[pl.ds(s * rb, rb)],
                            o0_hbm.at[pl.ds(s * rb, rb)])

        @pl.when(c == 1)
        def _():
            pltpu.sync_copy(acc.at[pl.ds(s * rb, rb)],
                            o1_hbm.at[pl.ds(s * rb, rb)])

    return k(xs0, xs1, rowp, colm)


def _dis(deg_blk):
    return jnp.where(deg_blk > 0.0,
                     lax.rsqrt(jnp.maximum(deg_blk, 1e-12)), 0.0)


def _tc_matmul(a, w):
    n, kk = a.shape
    m = w.shape[1]

    def body(a_ref, w_ref, o_ref):
        o_ref[...] = jnp.dot(a_ref[...], w_ref[...],
                             preferred_element_type=jnp.float32)

    return pl.pallas_call(
        body,
        grid=(n // BR,),
        in_specs=[pl.BlockSpec((BR, kk), lambda i: (i, 0)),
                  pl.BlockSpec((kk, m), lambda i: (0, 0))],
        out_specs=pl.BlockSpec((BR, m), lambda i: (i, 0)),
        out_shape=jax.ShapeDtypeStruct((n, m), jnp.float32),
    )(a, w)


def _tc_prescale(degs, x):
    n, f = x.shape

    def body(deg_ref, x_ref, o0_ref, o1_ref):
        dis = _dis(deg_ref[...])
        xb = x_ref[...]
        o0_ref[...] = xb[:, :HALF] * dis
        o1_ref[...] = xb[:, HALF:] * dis

    return pl.pallas_call(
        body,
        grid=(n // BR,),
        in_specs=[pl.BlockSpec((BR, HALF), lambda i: (i, 0)),
                  pl.BlockSpec((BR, f), lambda i: (i, 0))],
        out_specs=[pl.BlockSpec((BR, HALF), lambda i: (i, 0)),
                   pl.BlockSpec((BR, HALF), lambda i: (i, 0))],
        out_shape=[jax.ShapeDtypeStruct((n, HALF), jnp.float32),
                   jax.ShapeDtypeStruct((n, HALF), jnp.float32)],
    )(degs, x)


def _tc_mid(degs, out0, t0, t1, w11, b1r, w21):
    n, hid = out0.shape
    f = w21.shape[1]

    def body(deg_ref, o0_ref, t0_ref, t1_ref, w11a_ref, w11b_ref, b1_ref,
             w21_ref, h_ref, y0_ref, y1_ref):
        dis = _dis(deg_ref[...])
        a0 = t0_ref[...] * (-dis)
        a1 = t1_ref[...] * (-dis)
        h = (o0_ref[...]
             + jnp.dot(a0, w11a_ref[...], preferred_element_type=jnp.float32)
             + jnp.dot(a1, w11b_ref[...], preferred_element_type=jnp.float32)
             + b1_ref[...])
        h = jnp.maximum(h, 0.0)
        h_ref[...] = h
        y = jnp.dot(h, w21_ref[...], preferred_element_type=jnp.float32)
        y0_ref[...] = y[:, :HALF] * dis
        y1_ref[...] = y[:, HALF:] * dis

    return pl.pallas_call(
        body,
        grid=(n // BR,),
        in_specs=[pl.BlockSpec((BR, HALF), lambda i: (i, 0)),
                  pl.BlockSpec((BR, hid), lambda i: (i, 0)),
                  pl.BlockSpec((BR, HALF), lambda i: (i, 0)),
                  pl.BlockSpec((BR, HALF), lambda i: (i, 0)),
                  pl.BlockSpec((HALF, hid), lambda i: (0, 0)),
                  pl.BlockSpec((HALF, hid), lambda i: (1, 0)),
                  pl.BlockSpec((1, hid), lambda i: (0, 0)),
                  pl.BlockSpec((hid, f), lambda i: (0, 0))],
        out_specs=[pl.BlockSpec((BR, hid), lambda i: (i, 0)),
                   pl.BlockSpec((BR, HALF), lambda i: (i, 0)),
                   pl.BlockSpec((BR, HALF), lambda i: (i, 0))],
        out_shape=[jax.ShapeDtypeStruct((n, hid), jnp.float32),
                   jax.ShapeDtypeStruct((n, HALF), jnp.float32),
                   jax.ShapeDtypeStruct((n, HALF), jnp.float32)],
    )(degs, out0, t0, t1, w11, w11, b1r, w21)


def _tc_final(degs, z, u0, u1, b2r):
    n, f = z.shape

    def body(deg_ref, z_ref, u0_ref, u1_ref, b2_ref, o_ref):
        dis = _dis(deg_ref[...])
        u = jnp.concatenate([u0_ref[...] * dis, u1_ref[...] * dis], axis=1)
        o_ref[...] = jnp.maximum(z_ref[...] - u + b2_ref[...], 0.0)

    return pl.pallas_call(
        body,
        grid=(n // BR,),
        in_specs=[pl.BlockSpec((BR, HALF), lambda i: (i, 0)),
                  pl.BlockSpec((BR, f), lambda i: (i, 0)),
                  pl.BlockSpec((BR, HALF), lambda i: (i, 0)),
                  pl.BlockSpec((BR, HALF), lambda i: (i, 0)),
                  pl.BlockSpec((1, f), lambda i: (0, 0))],
        out_specs=pl.BlockSpec((BR, f), lambda i: (i, 0)),
        out_shape=jax.ShapeDtypeStruct((n, f), jnp.float32),
    )(degs, z, u0, u1, b2r)


def kernel(x, edge_index, W1_0, W1_1, b1, W2_0, W2_1, b2):
    n = x.shape[0]
    e = edge_index.shape[1]
    ew = NSUB * CHUNK
    ep = ((e + ew - 1) // ew) * ew
    rowp = jnp.concatenate(
        [edge_index[0], jnp.zeros((ep - e,), jnp.int32)])
    colp = jnp.concatenate(
        [edge_index[1], jnp.zeros((ep - e,), jnp.int32)])
    b1r = b1.reshape(1, -1)
    b2r = b2.reshape(1, -1)

    degs, colm = _sc_deg_colm(rowp, colp)
    out0 = _tc_matmul(x, W1_0)
    xs0, xs1 = _tc_prescale(degs, x)
    t0, t1 = _sc_prop(xs0, xs1, rowp, colm)
    h, ys0, ys1 = _tc_mid(degs, out0, t0, t1, W1_1, b1r, W2_1)
    z = _tc_matmul(h, W2_0)
    u0, u1 = _sc_prop(ys0, ys1, rowp, colm)
    return _tc_final(degs, z, u0, u1, b2r)
